# baseline (speedup 1.0000x reference)
import functools

import jax
import jax.numpy as jnp
import numpy as np
from jax import lax
from jax.experimental import pallas as pl
from jax.experimental.pallas import tpu as pltpu
from jax.experimental.pallas import tpu_sc as plsc

B = 4096
F = 26
V = 100000
E = 32
D = 128
NUM_TOKENS = 8
SEED = 0

_perm = np.random.RandomState(SEED).permutation(F)
_GROUPS = [
    _perm[F * i // NUM_TOKENS: F * (i + 1) // NUM_TOKENS].tolist()
    for i in range(NUM_TOKENS)
]

_GS = np.zeros((F, NUM_TOKENS), dtype=np.float32)
for _t, _g in enumerate(_GROUPS):
    for _f in _g:
        _GS[_f, _t] = 1.0 / len(_g)

_G = np.zeros((F * E, NUM_TOKENS * E), dtype=np.float32)
for _t, _g in enumerate(_GROUPS):
    for _f in _g:
        for _e in range(E):
            _G[_f * E + _e, _t * E + _e] = 1.0 / len(_g)

_NC, _NS = 2, 16
_NW = _NC * _NS
_BF = B * F
_RPW = _BF // _NW
_CH = 128
_NCH = _RPW // _CH
_RING = 6
_AHEAD = 4
_VROWS = F * V * E // 128


def _sc_gather_body(idx4_hbm, table_hbm, out_hbm, idx4_v, rows_v, gsem, wsem):
    wid = lax.axis_index("s") * _NC + lax.axis_index("c")
    pltpu.sync_copy(idx4_hbm.at[pl.ds(wid * 32, 32)], idx4_v)

    def gather(j):
        return pltpu.make_async_copy(
            table_hbm.at[idx4_v.at[j]],
            rows_v.at[j % _RING],
            gsem.at[j % _RING],
        )

    def write(j):
        return pltpu.make_async_copy(
            rows_v.at[j % _RING],
            out_hbm.at[pl.ds(wid * _RPW + j * _CH, _CH)],
            wsem.at[j % _RING],
        )

    for j in range(_AHEAD):
        gather(j).start()
    for j in range(_NCH):
        gather(j).wait()
        write(j).start()
        k = j + _AHEAD
        if k < _NCH:
            if k - _RING >= 0:
                write(k - _RING).wait()
            gather(k).start()
    for j in range(_NCH - _RING, _NCH):
        write(j).wait()


@functools.cache
def _sc_gather():
    return pl.kernel(
        _sc_gather_body,
        mesh=plsc.VectorSubcoreMesh(
            core_axis_name="c", subcore_axis_name="s", num_cores=_NC
        ),
        out_type=jax.ShapeDtypeStruct((_BF, 128), jnp.float32),
        scratch_types=[
            pltpu.VMEM((32, _CH), jnp.int32),
            pltpu.VMEM((_RING, _CH, 128), jnp.float32),
            pltpu.SemaphoreType.DMA((_RING,)),
            pltpu.SemaphoreType.DMA((_RING,)),
        ],
    )


def _tc_body(feats_ref, mask_ref, g_ref, p_ref, w_ref, b_ref, gamma_ref,
             beta_ref, out_ref):
    x = jnp.dot(feats_ref[...], g_ref[...], preferred_element_type=jnp.float32)
    x = x + jnp.dot(mask_ref[...], p_ref[...],
                    preferred_element_type=jnp.float32)
    gamma = gamma_ref[...]
    beta = beta_ref[...]
    bias = b_ref[...]
    for t in range(NUM_TOKENS):
        xt = x[:, t * E:(t + 1) * E]
        y = jnp.dot(xt, w_ref[...], preferred_element_type=jnp.float32) + bias
        y = y * jax.nn.sigmoid(y)
        mu = jnp.mean(y, axis=-1, keepdims=True)
        var = jnp.mean((y - mu) ** 2, axis=-1, keepdims=True)
        out_ref[:, t, :] = (y - mu) * lax.rsqrt(var + 1e-5) * gamma + beta


_TC_BLOCK = 512


def kernel(int_feats, missing_mask, tables, missing_emb, W, b, gamma, beta):
    offsets = (jnp.arange(F, dtype=jnp.int32) * V)[None, :]
    idx = (int_feats + offsets).reshape(_BF)
    idx4 = jnp.pad((idx >> 2).reshape(_NW, _NCH, _CH),
                   ((0, 0), (0, 32 - _NCH), (0, 0))).reshape(_NW * 32, _CH)
    table128 = tables.reshape(_VROWS, 128)

    raw = _sc_gather()(idx4, table128)

    sub = (idx & 3).reshape(_BF, 1)
    sel = jnp.arange(128, dtype=jnp.int32)[None, :] - sub * 32
    feats_flat = jnp.where((sel >= 0) & (sel < 32), raw, 0.0)
    feats_flat = (feats_flat[:, 0:32] + feats_flat[:, 32:64]
                  + feats_flat[:, 64:96] + feats_flat[:, 96:128])
    feats2d = feats_flat.reshape(B, F * E)

    p_mat = (jnp.asarray(_GS)[:, :, None] * missing_emb[0][:, None, :]
             ).reshape(F, NUM_TOKENS * E)

    grid = B // _TC_BLOCK
    out = pl.pallas_call(
        _tc_body,
        grid=(grid,),
        in_specs=[
            pl.BlockSpec((_TC_BLOCK, F * E), lambda i: (i, 0)),
            pl.BlockSpec((_TC_BLOCK, F), lambda i: (i, 0)),
            pl.BlockSpec((F * E, NUM_TOKENS * E), lambda i: (0, 0)),
            pl.BlockSpec((F, NUM_TOKENS * E), lambda i: (0, 0)),
            pl.BlockSpec((E, D), lambda i: (0, 0)),
            pl.BlockSpec((1, D), lambda i: (0, 0)),
            pl.BlockSpec((1, D), lambda i: (0, 0)),
            pl.BlockSpec((1, D), lambda i: (0, 0)),
        ],
        out_specs=pl.BlockSpec((_TC_BLOCK, NUM_TOKENS, D), lambda i: (i, 0, 0)),
        out_shape=jax.ShapeDtypeStruct((B, NUM_TOKENS, D), jnp.float32),
    )(feats2d, missing_mask, jnp.asarray(_G), p_mat, W,
      b.reshape(1, D), gamma.reshape(1, D), beta.reshape(1, D))
    return out
